# Initial kernel scaffold; baseline (speedup 1.0000x reference)
import jax
import jax.numpy as jnp
from jax.experimental import pallas as pl


def kernel(x, edge_index, batch, W1, b1, Wc0, bc0, Wc1, bc1, W2, b2, W3, b3, Wc, bc):
    raise NotImplementedError("write your pallas kernel here")


# speedup vs baseline: 9.2857x; 9.2857x over previous
import functools

import jax
import jax.numpy as jnp
from jax import lax
from jax.experimental import pallas as pl
from jax.experimental.pallas import tpu as pltpu
from jax.experimental.pallas import tpu_sc as plsc

N = 10000
E = 320000
DIM = 128
DIM2 = 64
DIM4 = 32
NUM_CLASSES = 16

NC = 2
NS = 16
NW = NC * NS
EPW = E // NW
G = 80
NB = EPW // G
NP = 10112
NPS = NP // NS
DW = 16

_mesh = lambda: plsc.VectorSubcoreMesh(
    core_axis_name="c", subcore_axis_name="s", num_cores=NC, num_subcores=NS)


def _sc_degree(dst3, ones_col, zeros_col):
  @functools.partial(
      pl.kernel,
      out_type=jax.ShapeDtypeStruct((NC, NP, DW), jnp.float32),
      mesh=_mesh(),
      scratch_types=[
          pltpu.VMEM((NB, G), jnp.int32),
          pltpu.VMEM((G, DW), jnp.float32),
          pltpu.VMEM_SHARED((NP, DW), jnp.float32),
      ],
      compiler_params=pltpu.CompilerParams(use_tc_tiling_on_sc=False),
  )
  def k(dst_hbm, ones_hbm, zeros_hbm, out_hbm, dst_v, ones_v, acc):
    c = lax.axis_index("c")
    s = lax.axis_index("s")
    wid = s * NC + c
    pltpu.sync_copy(dst_hbm.at[wid], dst_v)
    pltpu.sync_copy(ones_hbm, ones_v)
    pltpu.sync_copy(zeros_hbm, acc.at[pl.ds(s * NPS, NPS)])
    plsc.subcore_barrier()

    def body(j, carry):
      pltpu.sync_copy(ones_v, acc.at[dst_v.at[j]], add=True)
      return carry

    lax.fori_loop(0, NB, body, 0)
    plsc.subcore_barrier()
    pltpu.sync_copy(acc.at[pl.ds(s * NPS, NPS)],
                    out_hbm.at[c, pl.ds(s * NPS, NPS)])

  return k(dst3, ones_col, zeros_col)


def _sc_scatter(xs, src3, dst3, zeros_rows):
  @functools.partial(
      pl.kernel,
      out_type=jax.ShapeDtypeStruct((NC, NP, DIM), jnp.float32),
      mesh=_mesh(),
      scratch_types=[
          pltpu.VMEM((NB, G), jnp.int32),
          pltpu.VMEM((NB, G), jnp.int32),
          pltpu.VMEM((G, DIM), jnp.float32),
          pltpu.VMEM_SHARED((NP, DIM), jnp.float32),
      ],
  )
  def k(xs_hbm, src_hbm, dst_hbm, zeros_hbm, out_hbm, src_v, dst_v, rows, acc):
    c = lax.axis_index("c")
    s = lax.axis_index("s")
    wid = s * NC + c
    pltpu.sync_copy(src_hbm.at[wid], src_v)
    pltpu.sync_copy(dst_hbm.at[wid], dst_v)
    pltpu.sync_copy(zeros_hbm, acc.at[pl.ds(s * NPS, NPS)])
    plsc.subcore_barrier()

    def body(j, carry):
      pltpu.sync_copy(xs_hbm.at[src_v.at[j]], rows)
      pltpu.sync_copy(rows, acc.at[dst_v.at[j]], add=True)
      return carry

    lax.fori_loop(0, NB, body, 0)
    plsc.subcore_barrier()
    pltpu.sync_copy(acc.at[pl.ds(s * NPS, NPS)],
                    out_hbm.at[c, pl.ds(s * NPS, NPS)])

  return k(xs, src3, dst3, zeros_rows)


def _sc_edge_outputs(h, qt, qb, src3, dst3):
  @functools.partial(
      pl.kernel,
      out_type=(
          jax.ShapeDtypeStruct((E, 2 * DIM4), jnp.float32),
          jax.ShapeDtypeStruct((E, NUM_CLASSES), jnp.float32),
      ),
      mesh=_mesh(),
      scratch_types=[
          pltpu.VMEM((NB, G), jnp.int32),
          pltpu.VMEM((NB, G), jnp.int32),
          pltpu.VMEM((G, DIM4), jnp.float32),
          pltpu.VMEM((G, DIM4), jnp.float32),
          pltpu.VMEM((G, NUM_CLASSES), jnp.float32),
          pltpu.VMEM((G, NUM_CLASSES), jnp.float32),
          pltpu.VMEM((G, NUM_CLASSES), jnp.float32),
          pltpu.VMEM((G, 2 * DIM4), jnp.float32),
      ],
      compiler_params=pltpu.CompilerParams(use_tc_tiling_on_sc=False),
  )
  def k(h_hbm, qt_hbm, qb_hbm, src_hbm, dst_hbm, e_hbm, out_hbm,
        src_v, dst_v, hs, hd, qs, qd, ov, ev):
    c = lax.axis_index("c")
    s = lax.axis_index("s")
    wid = s * NC + c
    base = wid * EPW
    pltpu.sync_copy(src_hbm.at[wid], src_v)
    pltpu.sync_copy(dst_hbm.at[wid], dst_v)

    def body(j, carry):
      off = base + j * G
      pltpu.sync_copy(h_hbm.at[src_v.at[j]], hs)
      pltpu.sync_copy(h_hbm.at[dst_v.at[j]], hd)
      pltpu.sync_copy(qt_hbm.at[src_v.at[j]], qs)
      pltpu.sync_copy(qb_hbm.at[dst_v.at[j]], qd)
      L = 16
      for r in range(G):
        for v in range(DIM4 // L):
          ev[r, pl.ds(v * L, L)] = hs[r, pl.ds(v * L, L)]
          ev[r, pl.ds(DIM4 + v * L, L)] = hd[r, pl.ds(v * L, L)]
        ov[r, :] = qs[r, :] + qd[r, :]
      pltpu.sync_copy(ev, e_hbm.at[pl.ds(off, G)])
      pltpu.sync_copy(ov, out_hbm.at[pl.ds(off, G)])
      return carry

    lax.fori_loop(0, NB, body, 0)

  return k(h, qt, qb, src3, dst3)


def _tc_first(x, W1, degp):
  def body(x_ref, w_ref, degp_ref, xs_ref, dinv_ref):
    dinv = lax.rsqrt(degp_ref[0, :N, :1] + degp_ref[1, :N, :1] + 1.0)
    dinv_ref[...] = dinv
    xs_ref[...] = dinv * jnp.dot(x_ref[...], w_ref[...],
                                 preferred_element_type=jnp.float32)

  return pl.pallas_call(
      body,
      out_shape=(
          jax.ShapeDtypeStruct((N, DIM), jnp.float32),
          jax.ShapeDtypeStruct((N, 1), jnp.float32),
      ),
  )(x, W1, degp)


def _tc_layer(p, xs, dinv, b, Wn):
  def body(p_ref, xs_ref, dinv_ref, b_ref, w_ref, o_ref):
    h = jnp.tanh(dinv_ref[...] * (p_ref[0, :N] + p_ref[1, :N] + xs_ref[...])
                 + b_ref[...])
    o_ref[...] = dinv_ref[...] * jnp.dot(h, w_ref[...],
                                         preferred_element_type=jnp.float32)

  return pl.pallas_call(
      body,
      out_shape=jax.ShapeDtypeStruct((N, DIM), jnp.float32),
  )(p, xs, dinv, b, Wn)


def _tc_final(p, xs, dinv, bc1, W2, b2, W3, b3, Wct, Wcb, bc):
  def body(p_ref, xs_ref, dinv_ref, bc1_ref, w2_ref, b2_ref, w3_ref, b3_ref,
           wct_ref, wcb_ref, bc_ref, h_ref, qt_ref, qb_ref):
    h3 = jnp.tanh(dinv_ref[...] * (p_ref[0, :N] + p_ref[1, :N] + xs_ref[...])
                  + bc1_ref[...])
    h4 = jnp.tanh(jnp.dot(h3, w2_ref[...],
                          preferred_element_type=jnp.float32) + b2_ref[...])
    h5 = jnp.tanh(jnp.dot(h4, w3_ref[...],
                          preferred_element_type=jnp.float32) + b3_ref[...])
    h_ref[...] = h5
    qt_ref[...] = jnp.dot(h5, wct_ref[...],
                          preferred_element_type=jnp.float32) + bc_ref[...]
    qb_ref[...] = jnp.dot(h5, wcb_ref[...],
                          preferred_element_type=jnp.float32)

  return pl.pallas_call(
      body,
      out_shape=(
          jax.ShapeDtypeStruct((N, DIM4), jnp.float32),
          jax.ShapeDtypeStruct((N, NUM_CLASSES), jnp.float32),
          jax.ShapeDtypeStruct((N, NUM_CLASSES), jnp.float32),
      ),
  )(p, xs, dinv, bc1, W2, b2, W3, b3, Wct, Wcb, bc)


_DBG_XLA_DEG = False
_DBG_XLA_SCATTER = False
_DBG_XLA_EDGE = False


def _xla_deg(dst):
  d = jax.ops.segment_sum(jnp.ones((E,), jnp.float32), dst, num_segments=NP)
  return jnp.broadcast_to(jnp.stack([d, jnp.zeros_like(d)])[:, :, None],
                          (NC, NP, DW))


def _xla_scatter(xs, src, dst):
  s = jax.ops.segment_sum(xs[src], dst, num_segments=NP)
  return jnp.stack([s, jnp.zeros_like(s)])


def kernel(x, edge_index, batch, W1, b1, Wc0, bc0, Wc1, bc1, W2, b2, W3, b3,
           Wc, bc):
  del batch
  src3 = edge_index[0].reshape(NW, NB, G)
  dst3 = edge_index[1].reshape(NW, NB, G)
  ones_col = jnp.ones((G, DW), jnp.float32)
  zeros_col = jnp.zeros((NPS, DW), jnp.float32)
  zeros_rows = jnp.zeros((NPS, DIM), jnp.float32)

  if _DBG_XLA_DEG:
    degp = _xla_deg(edge_index[1])
  else:
    degp = _sc_degree(dst3, ones_col, zeros_col)
  xs1, dinv = _tc_first(x, W1, degp)

  def _scat(xs):
    if _DBG_XLA_SCATTER:
      return _xla_scatter(xs, edge_index[0], edge_index[1])
    return _sc_scatter(xs, src3, dst3, zeros_rows)

  p1 = _scat(xs1)
  xs2 = _tc_layer(p1, xs1, dinv, b1.reshape(1, DIM), Wc0)

  p2 = _scat(xs2)
  xs3 = _tc_layer(p2, xs2, dinv, bc0.reshape(1, DIM), Wc1)

  p3 = _scat(xs3)
  h5, qt, qb = _tc_final(p3, xs3, dinv, bc1.reshape(1, DIM), W2,
                         b2.reshape(1, DIM2), W3, b3.reshape(1, DIM4),
                         Wc[:DIM4], Wc[DIM4:], bc.reshape(1, NUM_CLASSES))

  if _DBG_XLA_EDGE:
    e = jnp.concatenate([h5[edge_index[0]], h5[edge_index[1]]], axis=1)
    out = qt[edge_index[0]] + qb[edge_index[1]]
  else:
    e, out = _sc_edge_outputs(h5, qt, qb, src3, dst3)
  return (out, e)


# speedup vs baseline: 10.3138x; 1.1107x over previous
import functools

import jax
import jax.numpy as jnp
from jax import lax
from jax.experimental import pallas as pl
from jax.experimental.pallas import tpu as pltpu
from jax.experimental.pallas import tpu_sc as plsc

N = 10000
E = 320000
DIM = 128
DIM2 = 64
DIM4 = 32
NUM_CLASSES = 16

NC = 2
NS = 16
NW = NC * NS
EPW = E // NW
G = 80
NB = EPW // G
NP = 10112
NPS = NP // NS
DW = 16

_mesh = lambda: plsc.VectorSubcoreMesh(
    core_axis_name="c", subcore_axis_name="s", num_cores=NC, num_subcores=NS)


def _sc_degree(dst3, ones_col, zeros_col):
  @functools.partial(
      pl.kernel,
      out_type=jax.ShapeDtypeStruct((NC, NP, DW), jnp.float32),
      mesh=_mesh(),
      scratch_types=[
          pltpu.VMEM((NB, G), jnp.int32),
          pltpu.VMEM((G, DW), jnp.float32),
          pltpu.VMEM_SHARED((NP, DW), jnp.float32),
      ],
      compiler_params=pltpu.CompilerParams(use_tc_tiling_on_sc=False),
  )
  def k(dst_hbm, ones_hbm, zeros_hbm, out_hbm, dst_v, ones_v, acc):
    c = lax.axis_index("c")
    s = lax.axis_index("s")
    wid = s * NC + c
    pltpu.sync_copy(dst_hbm.at[wid], dst_v)
    pltpu.sync_copy(ones_hbm, ones_v)
    pltpu.sync_copy(zeros_hbm, acc.at[pl.ds(s * NPS, NPS)])
    plsc.subcore_barrier()

    def body(j, carry):
      pltpu.sync_copy(ones_v, acc.at[dst_v.at[j]], add=True)
      return carry

    lax.fori_loop(0, NB, body, 0)
    plsc.subcore_barrier()
    pltpu.sync_copy(acc.at[pl.ds(s * NPS, NPS)],
                    out_hbm.at[c, pl.ds(s * NPS, NPS)])

  return k(dst3, ones_col, zeros_col)


def _sc_scatter(xs, src3, dst3, zeros_rows):
  @functools.partial(
      pl.kernel,
      out_type=jax.ShapeDtypeStruct((NC, NP, DIM), jnp.float32),
      mesh=_mesh(),
      scratch_types=[
          pltpu.VMEM((NB, G), jnp.int32),
          pltpu.VMEM((NB, G), jnp.int32),
          pltpu.VMEM((2, G, DIM), jnp.float32),
          pltpu.VMEM_SHARED((NP, DIM), jnp.float32),
          pltpu.SemaphoreType.DMA((2,)),
          pltpu.SemaphoreType.DMA((2,)),
      ],
      compiler_params=pltpu.CompilerParams(use_tc_tiling_on_sc=False),
  )
  def k(xs_hbm, src_hbm, dst_hbm, zeros_hbm, out_hbm, src_v, dst_v, rows, acc,
        sg, ss):
    c = lax.axis_index("c")
    s = lax.axis_index("s")
    wid = s * NC + c
    pltpu.sync_copy(src_hbm.at[wid], src_v)
    pltpu.sync_copy(dst_hbm.at[wid], dst_v)
    pltpu.sync_copy(zeros_hbm, acc.at[pl.ds(s * NPS, NPS)])
    plsc.subcore_barrier()

    def gather(j, b):
      pltpu.async_copy(xs_hbm.at[src_v.at[j]], rows.at[b], sg.at[b])

    def gather_wait(j, b):
      pltpu.make_async_copy(xs_hbm.at[src_v.at[j]], rows.at[b],
                            sg.at[b]).wait()

    def scat(j, b):
      pltpu.async_copy(rows.at[b], acc.at[dst_v.at[j]], ss.at[b], add=True)

    def scat_wait(j, b):
      pltpu.make_async_copy(rows.at[b], acc.at[dst_v.at[j]], ss.at[b]).wait()

    gather(0, 0)
    gather_wait(0, 0)
    gather(1, 1)
    scat(0, 0)

    def body(i, carry):
      b = i % 2
      gather_wait(i, b)
      scat_wait(i - 1, 1 - b)
      gather(i + 1, 1 - b)
      scat(i, b)
      return carry

    lax.fori_loop(1, NB - 1, body, 0)
    b_last = (NB - 1) % 2
    gather_wait(NB - 1, b_last)
    scat_wait(NB - 2, 1 - b_last)
    scat(NB - 1, b_last)
    scat_wait(NB - 1, b_last)
    plsc.subcore_barrier()
    pltpu.sync_copy(acc.at[pl.ds(s * NPS, NPS)],
                    out_hbm.at[c, pl.ds(s * NPS, NPS)])

  return k(xs, src3, dst3, zeros_rows)


def _sc_edge_outputs(h, qt, qb, src3, dst3):
  @functools.partial(
      pl.kernel,
      out_type=(
          jax.ShapeDtypeStruct((E, 2 * DIM4), jnp.float32),
          jax.ShapeDtypeStruct((E, NUM_CLASSES), jnp.float32),
      ),
      mesh=_mesh(),
      scratch_types=[
          pltpu.VMEM((NB, G), jnp.int32),
          pltpu.VMEM((NB, G), jnp.int32),
          pltpu.VMEM((G, DIM4), jnp.float32),
          pltpu.VMEM((G, DIM4), jnp.float32),
          pltpu.VMEM((G, NUM_CLASSES), jnp.float32),
          pltpu.VMEM((G, NUM_CLASSES), jnp.float32),
          pltpu.VMEM((G, NUM_CLASSES), jnp.float32),
          pltpu.VMEM((G, 2 * DIM4), jnp.float32),
      ],
      compiler_params=pltpu.CompilerParams(use_tc_tiling_on_sc=False),
  )
  def k(h_hbm, qt_hbm, qb_hbm, src_hbm, dst_hbm, e_hbm, out_hbm,
        src_v, dst_v, hs, hd, qs, qd, ov, ev):
    c = lax.axis_index("c")
    s = lax.axis_index("s")
    wid = s * NC + c
    base = wid * EPW
    pltpu.sync_copy(src_hbm.at[wid], src_v)
    pltpu.sync_copy(dst_hbm.at[wid], dst_v)

    def body(j, carry):
      off = base + j * G
      pltpu.sync_copy(h_hbm.at[src_v.at[j]], hs)
      pltpu.sync_copy(h_hbm.at[dst_v.at[j]], hd)
      pltpu.sync_copy(qt_hbm.at[src_v.at[j]], qs)
      pltpu.sync_copy(qb_hbm.at[dst_v.at[j]], qd)
      L = 16
      for r in range(G):
        for v in range(DIM4 // L):
          ev[r, pl.ds(v * L, L)] = hs[r, pl.ds(v * L, L)]
          ev[r, pl.ds(DIM4 + v * L, L)] = hd[r, pl.ds(v * L, L)]
        ov[r, :] = qs[r, :] + qd[r, :]
      pltpu.sync_copy(ev, e_hbm.at[pl.ds(off, G)])
      pltpu.sync_copy(ov, out_hbm.at[pl.ds(off, G)])
      return carry

    lax.fori_loop(0, NB, body, 0)

  return k(h, qt, qb, src3, dst3)


def _tc_first(x, W1, degp):
  def body(x_ref, w_ref, degp_ref, xs_ref, dinv_ref):
    dinv = lax.rsqrt(degp_ref[0, :N, :1] + degp_ref[1, :N, :1] + 1.0)
    dinv_ref[...] = dinv
    xs_ref[...] = dinv * jnp.dot(x_ref[...], w_ref[...],
                                 preferred_element_type=jnp.float32)

  return pl.pallas_call(
      body,
      out_shape=(
          jax.ShapeDtypeStruct((N, DIM), jnp.float32),
          jax.ShapeDtypeStruct((N, 1), jnp.float32),
      ),
  )(x, W1, degp)


def _tc_layer(p, xs, dinv, b, Wn):
  def body(p_ref, xs_ref, dinv_ref, b_ref, w_ref, o_ref):
    h = jnp.tanh(dinv_ref[...] * (p_ref[0, :N] + p_ref[1, :N] + xs_ref[...])
                 + b_ref[...])
    o_ref[...] = dinv_ref[...] * jnp.dot(h, w_ref[...],
                                         preferred_element_type=jnp.float32)

  return pl.pallas_call(
      body,
      out_shape=jax.ShapeDtypeStruct((N, DIM), jnp.float32),
  )(p, xs, dinv, b, Wn)


def _tc_final(p, xs, dinv, bc1, W2, b2, W3, b3, Wct, Wcb, bc):
  def body(p_ref, xs_ref, dinv_ref, bc1_ref, w2_ref, b2_ref, w3_ref, b3_ref,
           wct_ref, wcb_ref, bc_ref, h_ref, qt_ref, qb_ref):
    h3 = jnp.tanh(dinv_ref[...] * (p_ref[0, :N] + p_ref[1, :N] + xs_ref[...])
                  + bc1_ref[...])
    h4 = jnp.tanh(jnp.dot(h3, w2_ref[...],
                          preferred_element_type=jnp.float32) + b2_ref[...])
    h5 = jnp.tanh(jnp.dot(h4, w3_ref[...],
                          preferred_element_type=jnp.float32) + b3_ref[...])
    h_ref[...] = h5
    qt_ref[...] = jnp.dot(h5, wct_ref[...],
                          preferred_element_type=jnp.float32) + bc_ref[...]
    qb_ref[...] = jnp.dot(h5, wcb_ref[...],
                          preferred_element_type=jnp.float32)

  return pl.pallas_call(
      body,
      out_shape=(
          jax.ShapeDtypeStruct((N, DIM4), jnp.float32),
          jax.ShapeDtypeStruct((N, NUM_CLASSES), jnp.float32),
          jax.ShapeDtypeStruct((N, NUM_CLASSES), jnp.float32),
      ),
  )(p, xs, dinv, bc1, W2, b2, W3, b3, Wct, Wcb, bc)


_DBG_XLA_DEG = False
_DBG_XLA_SCATTER = False
_DBG_XLA_EDGE = False


def _xla_deg(dst):
  d = jax.ops.segment_sum(jnp.ones((E,), jnp.float32), dst, num_segments=NP)
  return jnp.broadcast_to(jnp.stack([d, jnp.zeros_like(d)])[:, :, None],
                          (NC, NP, DW))


def _xla_scatter(xs, src, dst):
  s = jax.ops.segment_sum(xs[src], dst, num_segments=NP)
  return jnp.stack([s, jnp.zeros_like(s)])


def kernel(x, edge_index, batch, W1, b1, Wc0, bc0, Wc1, bc1, W2, b2, W3, b3,
           Wc, bc):
  del batch
  src3 = edge_index[0].reshape(NW, NB, G)
  dst3 = edge_index[1].reshape(NW, NB, G)
  ones_col = jnp.ones((G, DW), jnp.float32)
  zeros_col = jnp.zeros((NPS, DW), jnp.float32)
  zeros_rows = jnp.zeros((NPS, DIM), jnp.float32)

  if _DBG_XLA_DEG:
    degp = _xla_deg(edge_index[1])
  else:
    degp = _sc_degree(dst3, ones_col, zeros_col)
  xs1, dinv = _tc_first(x, W1, degp)

  def _scat(xs):
    if _DBG_XLA_SCATTER:
      return _xla_scatter(xs, edge_index[0], edge_index[1])
    return _sc_scatter(xs, src3, dst3, zeros_rows)

  p1 = _scat(xs1)
  xs2 = _tc_layer(p1, xs1, dinv, b1.reshape(1, DIM), Wc0)

  p2 = _scat(xs2)
  xs3 = _tc_layer(p2, xs2, dinv, bc0.reshape(1, DIM), Wc1)

  p3 = _scat(xs3)
  h5, qt, qb = _tc_final(p3, xs3, dinv, bc1.reshape(1, DIM), W2,
                         b2.reshape(1, DIM2), W3, b3.reshape(1, DIM4),
                         Wc[:DIM4], Wc[DIM4:], bc.reshape(1, NUM_CLASSES))

  if _DBG_XLA_EDGE:
    e = jnp.concatenate([h5[edge_index[0]], h5[edge_index[1]]], axis=1)
    out = qt[edge_index[0]] + qb[edge_index[1]]
  else:
    e, out = _sc_edge_outputs(h5, qt, qb, src3, dst3)
  return (out, e)


# speedup vs baseline: 12.9935x; 1.2598x over previous
import functools

import jax
import jax.numpy as jnp
from jax import lax
from jax.experimental import pallas as pl
from jax.experimental.pallas import tpu as pltpu
from jax.experimental.pallas import tpu_sc as plsc

N = 10000
E = 320000
DIM = 128
DIM2 = 64
DIM4 = 32
NUM_CLASSES = 16

NC = 2
NS = 16
NW = NC * NS
EPW = E // NW
G = 80
NB = EPW // G
NP = 10112
NPS = NP // NS
DW = 16

_mesh = lambda: plsc.VectorSubcoreMesh(
    core_axis_name="c", subcore_axis_name="s", num_cores=NC, num_subcores=NS)


def _sc_degree(dst3, ones_col, zeros_col):
  @functools.partial(
      pl.kernel,
      out_type=jax.ShapeDtypeStruct((NC, NP, DW), jnp.float32),
      mesh=_mesh(),
      scratch_types=[
          pltpu.VMEM((NB, G), jnp.int32),
          pltpu.VMEM((G, DW), jnp.float32),
          pltpu.VMEM_SHARED((NP, DW), jnp.float32),
      ],
      compiler_params=pltpu.CompilerParams(use_tc_tiling_on_sc=False),
  )
  def k(dst_hbm, ones_hbm, zeros_hbm, out_hbm, dst_v, ones_v, acc):
    c = lax.axis_index("c")
    s = lax.axis_index("s")
    wid = s * NC + c
    pltpu.sync_copy(dst_hbm.at[wid], dst_v)
    pltpu.sync_copy(ones_hbm, ones_v)
    pltpu.sync_copy(zeros_hbm, acc.at[pl.ds(s * NPS, NPS)])
    plsc.subcore_barrier()

    def body(j, carry):
      pltpu.sync_copy(ones_v, acc.at[dst_v.at[j]], add=True)
      return carry

    lax.fori_loop(0, NB, body, 0)
    plsc.subcore_barrier()
    pltpu.sync_copy(acc.at[pl.ds(s * NPS, NPS)],
                    out_hbm.at[c, pl.ds(s * NPS, NPS)])

  return k(dst3, ones_col, zeros_col)


def _sc_scatter(xs, src3, dst3, zeros_rows):
  @functools.partial(
      pl.kernel,
      out_type=jax.ShapeDtypeStruct((NC, NP, DIM), jnp.float32),
      mesh=_mesh(),
      scratch_types=[
          pltpu.VMEM((NB, G), jnp.int32),
          pltpu.VMEM((NB, G), jnp.int32),
          pltpu.VMEM((2, G, DIM), jnp.float32),
          pltpu.VMEM_SHARED((NP, DIM), jnp.float32),
          pltpu.SemaphoreType.DMA((2,)),
          pltpu.SemaphoreType.DMA((2,)),
      ],
      compiler_params=pltpu.CompilerParams(use_tc_tiling_on_sc=False),
  )
  def k(xs_hbm, src_hbm, dst_hbm, zeros_hbm, out_hbm, src_v, dst_v, rows, acc,
        sg, ss):
    c = lax.axis_index("c")
    s = lax.axis_index("s")
    wid = s * NC + c
    pltpu.sync_copy(src_hbm.at[wid], src_v)
    pltpu.sync_copy(dst_hbm.at[wid], dst_v)
    pltpu.sync_copy(zeros_hbm, acc.at[pl.ds(s * NPS, NPS)])
    plsc.subcore_barrier()

    def gather(j, b):
      pltpu.async_copy(xs_hbm.at[src_v.at[j]], rows.at[b], sg.at[b])

    def gather_wait(j, b):
      pltpu.make_async_copy(xs_hbm.at[src_v.at[j]], rows.at[b],
                            sg.at[b]).wait()

    def scat(j, b):
      pltpu.async_copy(rows.at[b], acc.at[dst_v.at[j]], ss.at[b], add=True)

    def scat_wait(j, b):
      pltpu.make_async_copy(rows.at[b], acc.at[dst_v.at[j]], ss.at[b]).wait()

    gather(0, 0)
    gather_wait(0, 0)
    gather(1, 1)
    scat(0, 0)

    def body(i, carry):
      b = i % 2
      gather_wait(i, b)
      scat_wait(i - 1, 1 - b)
      gather(i + 1, 1 - b)
      scat(i, b)
      return carry

    lax.fori_loop(1, NB - 1, body, 0)
    b_last = (NB - 1) % 2
    gather_wait(NB - 1, b_last)
    scat_wait(NB - 2, 1 - b_last)
    scat(NB - 1, b_last)
    scat_wait(NB - 1, b_last)
    plsc.subcore_barrier()
    pltpu.sync_copy(acc.at[pl.ds(s * NPS, NPS)],
                    out_hbm.at[c, pl.ds(s * NPS, NPS)])

  return k(xs, src3, dst3, zeros_rows)


def _sc_edge_outputs(h, qt, qb, src3, dst3):
  @functools.partial(
      pl.kernel,
      out_type=(
          jax.ShapeDtypeStruct((E, 2 * DIM4), jnp.float32),
          jax.ShapeDtypeStruct((E, NUM_CLASSES), jnp.float32),
      ),
      mesh=_mesh(),
      scratch_types=[
          pltpu.VMEM((NB, G), jnp.int32),
          pltpu.VMEM((NB, G), jnp.int32),
          pltpu.VMEM((2, G, DIM4), jnp.float32),
          pltpu.VMEM((2, G, DIM4), jnp.float32),
          pltpu.VMEM((2, G, NUM_CLASSES), jnp.float32),
          pltpu.VMEM((2, G, NUM_CLASSES), jnp.float32),
          pltpu.VMEM((2, G, NUM_CLASSES), jnp.float32),
          pltpu.SemaphoreType.DMA((2,)),
          pltpu.SemaphoreType.DMA((2,)),
      ],
      compiler_params=pltpu.CompilerParams(use_tc_tiling_on_sc=False),
  )
  def k(h_hbm, qt_hbm, qb_hbm, src_hbm, dst_hbm, e_hbm, out_hbm,
        src_v, dst_v, hs, hd, qs, qd, ov, sgh, swr):
    c = lax.axis_index("c")
    s = lax.axis_index("s")
    wid = s * NC + c
    base = wid * EPW
    pltpu.sync_copy(src_hbm.at[wid], src_v)
    pltpu.sync_copy(dst_hbm.at[wid], dst_v)

    def g_copies(j, b):
      return (
          (h_hbm.at[src_v.at[j]], hs.at[b]),
          (h_hbm.at[dst_v.at[j]], hd.at[b]),
          (qt_hbm.at[src_v.at[j]], qs.at[b]),
          (qb_hbm.at[dst_v.at[j]], qd.at[b]),
      )

    def g(j, b):
      for s_ref, d_ref in g_copies(j, b):
        pltpu.async_copy(s_ref, d_ref, sgh.at[b])

    def g_wait(j, b):
      for s_ref, d_ref in g_copies(j, b):
        pltpu.make_async_copy(s_ref, d_ref, sgh.at[b]).wait()

    def w_copies(j, b):
      off = base + j * G
      return (
          (hs.at[b], e_hbm.at[pl.ds(off, G), pl.ds(0, DIM4)]),
          (hd.at[b], e_hbm.at[pl.ds(off, G), pl.ds(DIM4, DIM4)]),
          (ov.at[b], out_hbm.at[pl.ds(off, G)]),
      )

    def w(j, b):
      for s_ref, d_ref in w_copies(j, b):
        pltpu.async_copy(s_ref, d_ref, swr.at[b])

    def w_wait(j, b):
      for s_ref, d_ref in w_copies(j, b):
        pltpu.make_async_copy(s_ref, d_ref, swr.at[b]).wait()

    def compute(b):
      qsb, qdb, ovb = qs.at[b], qd.at[b], ov.at[b]
      for r in range(G):
        ovb[r, :] = qsb[r, :] + qdb[r, :]

    g(0, 0)
    g_wait(0, 0)
    g(1, 1)
    compute(0)
    w(0, 0)

    def body(i, carry):
      b = i % 2
      g_wait(i, b)
      w_wait(i - 1, 1 - b)
      g(i + 1, 1 - b)
      compute(b)
      w(i, b)
      return carry

    lax.fori_loop(1, NB - 1, body, 0)
    b_last = (NB - 1) % 2
    g_wait(NB - 1, b_last)
    w_wait(NB - 2, 1 - b_last)
    compute(b_last)
    w(NB - 1, b_last)
    w_wait(NB - 1, b_last)

  return k(h, qt, qb, src3, dst3)


def _tc_first(x, W1, degp):
  def body(x_ref, w_ref, degp_ref, xs_ref, dinv_ref):
    dinv = lax.rsqrt(degp_ref[0, :N, :1] + degp_ref[1, :N, :1] + 1.0)
    dinv_ref[...] = dinv
    xs_ref[...] = dinv * jnp.dot(x_ref[...], w_ref[...],
                                 preferred_element_type=jnp.float32)

  return pl.pallas_call(
      body,
      out_shape=(
          jax.ShapeDtypeStruct((N, DIM), jnp.float32),
          jax.ShapeDtypeStruct((N, 1), jnp.float32),
      ),
  )(x, W1, degp)


def _tc_layer(p, xs, dinv, b, Wn):
  def body(p_ref, xs_ref, dinv_ref, b_ref, w_ref, o_ref):
    h = jnp.tanh(dinv_ref[...] * (p_ref[0, :N] + p_ref[1, :N] + xs_ref[...])
                 + b_ref[...])
    o_ref[...] = dinv_ref[...] * jnp.dot(h, w_ref[...],
                                         preferred_element_type=jnp.float32)

  return pl.pallas_call(
      body,
      out_shape=jax.ShapeDtypeStruct((N, DIM), jnp.float32),
  )(p, xs, dinv, b, Wn)


def _tc_final(p, xs, dinv, bc1, W2, b2, W3, b3, Wct, Wcb, bc):
  def body(p_ref, xs_ref, dinv_ref, bc1_ref, w2_ref, b2_ref, w3_ref, b3_ref,
           wct_ref, wcb_ref, bc_ref, h_ref, qt_ref, qb_ref):
    h3 = jnp.tanh(dinv_ref[...] * (p_ref[0, :N] + p_ref[1, :N] + xs_ref[...])
                  + bc1_ref[...])
    h4 = jnp.tanh(jnp.dot(h3, w2_ref[...],
                          preferred_element_type=jnp.float32) + b2_ref[...])
    h5 = jnp.tanh(jnp.dot(h4, w3_ref[...],
                          preferred_element_type=jnp.float32) + b3_ref[...])
    h_ref[...] = h5
    qt_ref[...] = jnp.dot(h5, wct_ref[...],
                          preferred_element_type=jnp.float32) + bc_ref[...]
    qb_ref[...] = jnp.dot(h5, wcb_ref[...],
                          preferred_element_type=jnp.float32)

  return pl.pallas_call(
      body,
      out_shape=(
          jax.ShapeDtypeStruct((N, DIM4), jnp.float32),
          jax.ShapeDtypeStruct((N, NUM_CLASSES), jnp.float32),
          jax.ShapeDtypeStruct((N, NUM_CLASSES), jnp.float32),
      ),
  )(p, xs, dinv, bc1, W2, b2, W3, b3, Wct, Wcb, bc)


_DBG_XLA_DEG = False
_DBG_XLA_SCATTER = False
_DBG_XLA_EDGE = False


def _xla_deg(dst):
  d = jax.ops.segment_sum(jnp.ones((E,), jnp.float32), dst, num_segments=NP)
  return jnp.broadcast_to(jnp.stack([d, jnp.zeros_like(d)])[:, :, None],
                          (NC, NP, DW))


def _xla_scatter(xs, src, dst):
  s = jax.ops.segment_sum(xs[src], dst, num_segments=NP)
  return jnp.stack([s, jnp.zeros_like(s)])


def kernel(x, edge_index, batch, W1, b1, Wc0, bc0, Wc1, bc1, W2, b2, W3, b3,
           Wc, bc):
  del batch
  src3 = edge_index[0].reshape(NW, NB, G)
  dst3 = edge_index[1].reshape(NW, NB, G)
  ones_col = jnp.ones((G, DW), jnp.float32)
  zeros_col = jnp.zeros((NPS, DW), jnp.float32)
  zeros_rows = jnp.zeros((NPS, DIM), jnp.float32)

  if _DBG_XLA_DEG:
    degp = _xla_deg(edge_index[1])
  else:
    degp = _sc_degree(dst3, ones_col, zeros_col)
  xs1, dinv = _tc_first(x, W1, degp)

  def _scat(xs):
    if _DBG_XLA_SCATTER:
      return _xla_scatter(xs, edge_index[0], edge_index[1])
    return _sc_scatter(xs, src3, dst3, zeros_rows)

  p1 = _scat(xs1)
  xs2 = _tc_layer(p1, xs1, dinv, b1.reshape(1, DIM), Wc0)

  p2 = _scat(xs2)
  xs3 = _tc_layer(p2, xs2, dinv, bc0.reshape(1, DIM), Wc1)

  p3 = _scat(xs3)
  h5, qt, qb = _tc_final(p3, xs3, dinv, bc1.reshape(1, DIM), W2,
                         b2.reshape(1, DIM2), W3, b3.reshape(1, DIM4),
                         Wc[:DIM4], Wc[DIM4:], bc.reshape(1, NUM_CLASSES))

  if _DBG_XLA_EDGE:
    e = jnp.concatenate([h5[edge_index[0]], h5[edge_index[1]]], axis=1)
    out = qt[edge_index[0]] + qb[edge_index[1]]
  else:
    e, out = _sc_edge_outputs(h5, qt, qb, src3, dst3)
  return (out, e)


# speedup vs baseline: 13.0316x; 1.0029x over previous
import functools

import jax
import jax.numpy as jnp
from jax import lax
from jax.experimental import pallas as pl
from jax.experimental.pallas import tpu as pltpu
from jax.experimental.pallas import tpu_sc as plsc

N = 10000
E = 320000
DIM = 128
DIM2 = 64
DIM4 = 32
NUM_CLASSES = 16

NC = 2
NS = 16
NW = NC * NS
EPW = E // NW
G = 80
NB = EPW // G
NP = 10112
NPS = NP // NS
DW = 16

_mesh = lambda: plsc.VectorSubcoreMesh(
    core_axis_name="c", subcore_axis_name="s", num_cores=NC, num_subcores=NS)


def _sc_degree(dst3, ones_col, zeros_col):
  @functools.partial(
      pl.kernel,
      out_type=jax.ShapeDtypeStruct((NC, NP, DW), jnp.float32),
      mesh=_mesh(),
      scratch_types=[
          pltpu.VMEM((NB, G), jnp.int32),
          pltpu.VMEM((G, DW), jnp.float32),
          pltpu.VMEM_SHARED((NP, DW), jnp.float32),
          pltpu.SemaphoreType.DMA,
      ],
      compiler_params=pltpu.CompilerParams(use_tc_tiling_on_sc=False),
  )
  def k(dst_hbm, ones_hbm, zeros_hbm, out_hbm, dst_v, ones_v, acc, sem):
    c = lax.axis_index("c")
    s = lax.axis_index("s")
    wid = s * NC + c
    pltpu.sync_copy(dst_hbm.at[wid], dst_v)
    pltpu.sync_copy(ones_hbm, ones_v)
    pltpu.sync_copy(zeros_hbm, acc.at[pl.ds(s * NPS, NPS)])
    plsc.subcore_barrier()

    W = 8

    def scat(j):
      pltpu.async_copy(ones_v, acc.at[dst_v.at[j]], sem, add=True)

    def scat_wait(j):
      pltpu.make_async_copy(ones_v, acc.at[dst_v.at[j]], sem).wait()

    def body(j, carry):
      scat(j)
      scat_wait(j - W)
      return carry

    lax.fori_loop(0, W, lambda j, c_: (scat(j), c_)[1], 0)
    lax.fori_loop(W, NB, body, 0)
    lax.fori_loop(NB - W, NB, lambda j, c_: (scat_wait(j), c_)[1], 0)
    plsc.subcore_barrier()
    pltpu.sync_copy(acc.at[pl.ds(s * NPS, NPS)],
                    out_hbm.at[c, pl.ds(s * NPS, NPS)])

  return k(dst3, ones_col, zeros_col)


def _sc_scatter(xs, src3, dst3, zeros_rows):
  @functools.partial(
      pl.kernel,
      out_type=jax.ShapeDtypeStruct((NC, NP, DIM), jnp.float32),
      mesh=_mesh(),
      scratch_types=[
          pltpu.VMEM((NB, G), jnp.int32),
          pltpu.VMEM((NB, G), jnp.int32),
          pltpu.VMEM((2, G, DIM), jnp.float32),
          pltpu.VMEM_SHARED((NP, DIM), jnp.float32),
          pltpu.SemaphoreType.DMA((2,)),
          pltpu.SemaphoreType.DMA((2,)),
      ],
      compiler_params=pltpu.CompilerParams(use_tc_tiling_on_sc=False),
  )
  def k(xs_hbm, src_hbm, dst_hbm, zeros_hbm, out_hbm, src_v, dst_v, rows, acc,
        sg, ss):
    c = lax.axis_index("c")
    s = lax.axis_index("s")
    wid = s * NC + c
    pltpu.sync_copy(src_hbm.at[wid], src_v)
    pltpu.sync_copy(dst_hbm.at[wid], dst_v)
    pltpu.sync_copy(zeros_hbm, acc.at[pl.ds(s * NPS, NPS)])
    plsc.subcore_barrier()

    def gather(j, b):
      pltpu.async_copy(xs_hbm.at[src_v.at[j]], rows.at[b], sg.at[b])

    def gather_wait(j, b):
      pltpu.make_async_copy(xs_hbm.at[src_v.at[j]], rows.at[b],
                            sg.at[b]).wait()

    def scat(j, b):
      pltpu.async_copy(rows.at[b], acc.at[dst_v.at[j]], ss.at[b], add=True)

    def scat_wait(j, b):
      pltpu.make_async_copy(rows.at[b], acc.at[dst_v.at[j]], ss.at[b]).wait()

    gather(0, 0)
    gather_wait(0, 0)
    gather(1, 1)
    scat(0, 0)

    def body(i, carry):
      b = i % 2
      gather_wait(i, b)
      scat_wait(i - 1, 1 - b)
      gather(i + 1, 1 - b)
      scat(i, b)
      return carry

    lax.fori_loop(1, NB - 1, body, 0)
    b_last = (NB - 1) % 2
    gather_wait(NB - 1, b_last)
    scat_wait(NB - 2, 1 - b_last)
    scat(NB - 1, b_last)
    scat_wait(NB - 1, b_last)
    plsc.subcore_barrier()
    pltpu.sync_copy(acc.at[pl.ds(s * NPS, NPS)],
                    out_hbm.at[c, pl.ds(s * NPS, NPS)])

  return k(xs, src3, dst3, zeros_rows)


def _sc_edge_outputs(th, tb, src3, dst3):
  TW = DIM4 + NUM_CLASSES

  @functools.partial(
      pl.kernel,
      out_type=(
          jax.ShapeDtypeStruct((E, 2 * DIM4), jnp.float32),
          jax.ShapeDtypeStruct((E, NUM_CLASSES), jnp.float32),
      ),
      mesh=_mesh(),
      scratch_types=[
          pltpu.VMEM((NB, G), jnp.int32),
          pltpu.VMEM((NB, G), jnp.int32),
          pltpu.VMEM((2, G, TW), jnp.float32),
          pltpu.VMEM((2, G, TW), jnp.float32),
          pltpu.VMEM((2, G, NUM_CLASSES), jnp.float32),
          pltpu.SemaphoreType.DMA((2,)),
          pltpu.SemaphoreType.DMA((2,)),
      ],
      compiler_params=pltpu.CompilerParams(use_tc_tiling_on_sc=False),
  )
  def k(th_hbm, tb_hbm, src_hbm, dst_hbm, e_hbm, out_hbm,
        src_v, dst_v, bs, bd, ov, sgh, swr):
    c = lax.axis_index("c")
    s = lax.axis_index("s")
    wid = s * NC + c
    base = wid * EPW
    pltpu.sync_copy(src_hbm.at[wid], src_v)
    pltpu.sync_copy(dst_hbm.at[wid], dst_v)

    def g_copies(j, b):
      return (
          (th_hbm.at[src_v.at[j]], bs.at[b]),
          (tb_hbm.at[dst_v.at[j]], bd.at[b]),
      )

    def g(j, b):
      for s_ref, d_ref in g_copies(j, b):
        pltpu.async_copy(s_ref, d_ref, sgh.at[b])

    def g_wait(j, b):
      for s_ref, d_ref in g_copies(j, b):
        pltpu.make_async_copy(s_ref, d_ref, sgh.at[b]).wait()

    def w_copies(j, b):
      off = base + j * G
      return (
          (bs.at[b, :, pl.ds(0, DIM4)],
           e_hbm.at[pl.ds(off, G), pl.ds(0, DIM4)]),
          (bd.at[b, :, pl.ds(0, DIM4)],
           e_hbm.at[pl.ds(off, G), pl.ds(DIM4, DIM4)]),
          (ov.at[b], out_hbm.at[pl.ds(off, G)]),
      )

    def w(j, b):
      for s_ref, d_ref in w_copies(j, b):
        pltpu.async_copy(s_ref, d_ref, swr.at[b])

    def w_wait(j, b):
      for s_ref, d_ref in w_copies(j, b):
        pltpu.make_async_copy(s_ref, d_ref, swr.at[b]).wait()

    def compute(b):
      bsb, bdb, ovb = bs.at[b], bd.at[b], ov.at[b]
      for r in range(G):
        ovb[r, :] = (bsb[r, pl.ds(DIM4, NUM_CLASSES)]
                     + bdb[r, pl.ds(DIM4, NUM_CLASSES)])

    g(0, 0)
    g_wait(0, 0)
    g(1, 1)
    compute(0)
    w(0, 0)

    def body(i, carry):
      b = i % 2
      g_wait(i, b)
      w_wait(i - 1, 1 - b)
      g(i + 1, 1 - b)
      compute(b)
      w(i, b)
      return carry

    lax.fori_loop(1, NB - 1, body, 0)
    b_last = (NB - 1) % 2
    g_wait(NB - 1, b_last)
    w_wait(NB - 2, 1 - b_last)
    compute(b_last)
    w(NB - 1, b_last)
    w_wait(NB - 1, b_last)

  return k(th, tb, src3, dst3)


def _tc_first(x, W1, degp):
  def body(x_ref, w_ref, degp_ref, xs_ref, dinv_ref):
    dinv = lax.rsqrt(degp_ref[0, :N, :1] + degp_ref[1, :N, :1] + 1.0)
    dinv_ref[...] = dinv
    xs_ref[...] = dinv * jnp.dot(x_ref[...], w_ref[...],
                                 preferred_element_type=jnp.float32)

  return pl.pallas_call(
      body,
      out_shape=(
          jax.ShapeDtypeStruct((N, DIM), jnp.float32),
          jax.ShapeDtypeStruct((N, 1), jnp.float32),
      ),
  )(x, W1, degp)


def _tc_layer(p, xs, dinv, b, Wn):
  def body(p_ref, xs_ref, dinv_ref, b_ref, w_ref, o_ref):
    h = jnp.tanh(dinv_ref[...] * (p_ref[0, :N] + p_ref[1, :N] + xs_ref[...])
                 + b_ref[...])
    o_ref[...] = dinv_ref[...] * jnp.dot(h, w_ref[...],
                                         preferred_element_type=jnp.float32)

  return pl.pallas_call(
      body,
      out_shape=jax.ShapeDtypeStruct((N, DIM), jnp.float32),
  )(p, xs, dinv, b, Wn)


def _tc_final(p, xs, dinv, bc1, W2, b2, W3, b3, Wct, Wcb, bc):
  def body(p_ref, xs_ref, dinv_ref, bc1_ref, w2_ref, b2_ref, w3_ref, b3_ref,
           wct_ref, wcb_ref, bc_ref, th_ref, tb_ref):
    h3 = jnp.tanh(dinv_ref[...] * (p_ref[0, :N] + p_ref[1, :N] + xs_ref[...])
                  + bc1_ref[...])
    h4 = jnp.tanh(jnp.dot(h3, w2_ref[...],
                          preferred_element_type=jnp.float32) + b2_ref[...])
    h5 = jnp.tanh(jnp.dot(h4, w3_ref[...],
                          preferred_element_type=jnp.float32) + b3_ref[...])
    qt = jnp.dot(h5, wct_ref[...],
                 preferred_element_type=jnp.float32) + bc_ref[...]
    qb = jnp.dot(h5, wcb_ref[...], preferred_element_type=jnp.float32)
    th_ref[...] = jnp.concatenate([h5, qt], axis=1)
    tb_ref[...] = jnp.concatenate([h5, qb], axis=1)

  return pl.pallas_call(
      body,
      out_shape=(
          jax.ShapeDtypeStruct((N, DIM4 + NUM_CLASSES), jnp.float32),
          jax.ShapeDtypeStruct((N, DIM4 + NUM_CLASSES), jnp.float32),
      ),
  )(p, xs, dinv, bc1, W2, b2, W3, b3, Wct, Wcb, bc)


def kernel(x, edge_index, batch, W1, b1, Wc0, bc0, Wc1, bc1, W2, b2, W3, b3,
           Wc, bc):
  del batch
  src3 = edge_index[0].reshape(NW, NB, G)
  dst3 = edge_index[1].reshape(NW, NB, G)
  ones_col = jnp.ones((G, DW), jnp.float32)
  zeros_col = jnp.zeros((NPS, DW), jnp.float32)
  zeros_rows = jnp.zeros((NPS, DIM), jnp.float32)

  degp = _sc_degree(dst3, ones_col, zeros_col)
  xs1, dinv = _tc_first(x, W1, degp)

  p1 = _sc_scatter(xs1, src3, dst3, zeros_rows)
  xs2 = _tc_layer(p1, xs1, dinv, b1.reshape(1, DIM), Wc0)

  p2 = _sc_scatter(xs2, src3, dst3, zeros_rows)
  xs3 = _tc_layer(p2, xs2, dinv, bc0.reshape(1, DIM), Wc1)

  p3 = _sc_scatter(xs3, src3, dst3, zeros_rows)
  th, tb = _tc_final(p3, xs3, dinv, bc1.reshape(1, DIM), W2,
                     b2.reshape(1, DIM2), W3, b3.reshape(1, DIM4),
                     Wc[:DIM4], Wc[DIM4:], bc.reshape(1, NUM_CLASSES))

  e, out = _sc_edge_outputs(th, tb, src3, dst3)
  return (out, e)
